# Initial kernel scaffold; baseline (speedup 1.0000x reference)
import functools

import jax
import jax.numpy as jnp
from jax.experimental import pallas as pl
from jax.experimental.pallas import tpu as pltpu
from jax.experimental.pallas import tpu_sc as plsc

BATCH = 16384
HIST = 50
EMB = 64
NUM_IDX = BATCH * HIST

WINDOW = 128


def _sc_gather(idx, table):
    mesh = plsc.VectorSubcoreMesh(core_axis_name="c", subcore_axis_name="s")

    @functools.partial(
        pl.kernel,
        mesh=mesh,
        out_type=jax.ShapeDtypeStruct((NUM_IDX, EMB), jnp.float32),
    )
    def k(table_hbm, idx_hbm, out_hbm):
        def body(idx_v, out_v):
            pltpu.sync_copy(table_hbm.at[idx_v.at[0]], out_v)

        pltpu.emit_pipeline(
            body,
            grid=(NUM_IDX // WINDOW,),
            in_specs=[
                pl.BlockSpec((1, WINDOW), index_map=lambda i: (0, i)),
            ],
            out_specs=[
                pl.BlockSpec((WINDOW, EMB), index_map=lambda i: (i, 0)),
            ],
            core_axis_name=("c", "s"),
            dimension_semantics=(pltpu.PARALLEL,),
        )(idx_hbm, out_hbm)

    return k(table, idx.reshape(1, NUM_IDX))


def kernel(x, table):
    idx = x.reshape(-1).astype(jnp.int32)
    out = _sc_gather(idx, table)
    return out.reshape(BATCH, HIST, EMB)


# speedup vs baseline: 1.5735x; 1.5735x over previous
import functools

import jax
import jax.numpy as jnp
from jax import lax
from jax.experimental import pallas as pl
from jax.experimental.pallas import tpu as pltpu
from jax.experimental.pallas import tpu_sc as plsc

BATCH = 16384
HIST = 50
EMB = 64
NUM_IDX = BATCH * HIST

NW = 32
B_PER_W = NUM_IDX // NW
CH = 128
N_CHUNK = B_PER_W // CH


def _sc_gather(idx, table):
    mesh = plsc.VectorSubcoreMesh(core_axis_name="c", subcore_axis_name="s")

    @functools.partial(
        pl.kernel,
        mesh=mesh,
        out_type=jax.ShapeDtypeStruct((NUM_IDX, EMB), jnp.float32),
        compiler_params=pltpu.CompilerParams(use_tc_tiling_on_sc=False),
        scratch_types=[
            pltpu.VMEM((CH,), jnp.int32),
            pltpu.VMEM((CH, EMB), jnp.float32),
            pltpu.SemaphoreType.DMA,
        ],
    )
    def k(table_hbm, idx_hbm, out_hbm, idx_v, rows_v, sem):
        wid = lax.axis_index("s") * 2 + lax.axis_index("c")
        base = wid * B_PER_W

        @pl.loop(0, N_CHUNK)
        def _(g):
            off = base + g * CH
            pltpu.sync_copy(idx_hbm.at[pl.ds(off, CH)], idx_v)
            pltpu.async_copy(table_hbm.at[idx_v], rows_v, sem).wait()
            pltpu.sync_copy(rows_v, out_hbm.at[pl.ds(off, CH)])

    return k(table, idx)


def kernel(x, table):
    idx = x.reshape(-1).astype(jnp.int32)
    out = _sc_gather(idx, table)
    return out.reshape(BATCH, HIST, EMB)


# speedup vs baseline: 1.8779x; 1.1935x over previous
import functools

import jax
import jax.numpy as jnp
from jax import lax
from jax.experimental import pallas as pl
from jax.experimental.pallas import tpu as pltpu
from jax.experimental.pallas import tpu_sc as plsc

BATCH = 16384
HIST = 50
EMB = 64
NUM_IDX = BATCH * HIST

NW = 32
B_PER_W = NUM_IDX // NW
GW = 128
SUP = 512
KG = SUP // GW
N_SUP = B_PER_W // SUP


def _sc_gather(idx, table):
    mesh = plsc.VectorSubcoreMesh(core_axis_name="c", subcore_axis_name="s")

    @functools.partial(
        pl.kernel,
        mesh=mesh,
        out_type=jax.ShapeDtypeStruct((NUM_IDX, EMB), jnp.float32),
        compiler_params=pltpu.CompilerParams(use_tc_tiling_on_sc=False),
        scratch_types=[
            pltpu.VMEM((B_PER_W,), jnp.int32),
            pltpu.VMEM((2, SUP, EMB), jnp.float32),
            pltpu.SemaphoreType.DMA((2,)),
            pltpu.SemaphoreType.DMA((2,)),
        ],
    )
    def k(table_hbm, idx_hbm, out_hbm, idx_v, rows_v, gsem, wsem):
        wid = lax.axis_index("s") * 2 + lax.axis_index("c")
        base = wid * B_PER_W

        pltpu.sync_copy(idx_hbm.at[pl.ds(base, B_PER_W)], idx_v)

        def fire(slot, b):
            for j in range(KG):
                pltpu.async_copy(
                    table_hbm.at[idx_v.at[pl.ds(slot * SUP + j * GW, GW)]],
                    rows_v.at[b].at[pl.ds(j * GW, GW)],
                    gsem.at[b],
                )

        def drain_gathers(slot, b):
            for j in range(KG):
                pltpu.make_async_copy(
                    table_hbm.at[idx_v.at[pl.ds(slot * SUP + j * GW, GW)]],
                    rows_v.at[b].at[pl.ds(j * GW, GW)],
                    gsem.at[b],
                ).wait()

        def wait_writeback(b):
            pltpu.make_async_copy(
                rows_v.at[b],
                out_hbm.at[pl.ds(base, SUP)],
                wsem.at[b],
            ).wait()

        fire(0, 0)

        @pl.loop(0, N_SUP, step=2)
        def _(g):
            for b in range(2):
                slot = g + b
                oth = 1 - b
                nxt = slot + 1

                @pl.when(nxt < N_SUP)
                def _():
                    @pl.when(slot >= 1)
                    def _():
                        wait_writeback(oth)

                    fire(nxt, oth)

                drain_gathers(slot, b)
                pltpu.async_copy(
                    rows_v.at[b],
                    out_hbm.at[pl.ds(base + slot * SUP, SUP)],
                    wsem.at[b],
                )

        for b in range(2):
            wait_writeback(b)

    return k(table, idx)


def kernel(x, table):
    idx = x.reshape(-1).astype(jnp.int32)
    out = _sc_gather(idx, table)
    return out.reshape(BATCH, HIST, EMB)
